# Initial kernel scaffold; baseline (speedup 1.0000x reference)
import jax
import jax.numpy as jnp
from jax.experimental import pallas as pl


def kernel(dipoles, cell, positions, neighbor_indices, neighbor_vectors):
    raise NotImplementedError("write your pallas kernel here")


# speedup vs baseline: 33.6301x; 33.6301x over previous
import functools

import jax
import jax.numpy as jnp
from jax import lax
from jax.experimental import pallas as pl
from jax.experimental.pallas import tpu as pltpu
from jax.experimental.pallas import tpu_sc as plsc

N = 50000
E = 1_600_000
NC, NS, L = 2, 16, 16
NW = NC * NS
NPAD = 53248
SEG = NPAD // NS
B = 1024
SUB = B // 128
NBLK = 50
PER_W = NBLK * B
EPAD = NW * PER_W
NROWS = EPAD // 128
R2C = NPAD // NW
W2C = R2C * 3

_mesh = plsc.VectorSubcoreMesh(core_axis_name="c", subcore_axis_name="s")
_params = pltpu.CompilerParams(needs_layout_passes=False)


@functools.partial(
    pl.kernel,
    out_type=jax.ShapeDtypeStruct((NC * 3 * NPAD,), jnp.float32),
    mesh=_mesh,
    compiler_params=_params,
    scratch_types=(
        [pltpu.VMEM_SHARED((NPAD,), jnp.float32) for _ in range(3)]
        + [pltpu.VMEM_SHARED((NPAD,), jnp.float32) for _ in range(3)]
        + [pltpu.VMEM((SUB, 128), jnp.int32) for _ in range(2)]
        + [pltpu.VMEM((B,), jnp.float32) for _ in range(3)]
        + [pltpu.VMEM((B,), jnp.float32) for _ in range(6)]
        + [pltpu.VMEM((B,), jnp.float32) for _ in range(6)]
        + [pltpu.SemaphoreType.DMA, pltpu.SemaphoreType.DMA]
    ),
)
def _edge_kernel(dip_h, zeros_h, ii_h, jj_h, vx_h, vy_h, vz_h, out_h,
                 dsx, dsy, dsz, asx, asy, asz, ii_v, jj_v,
                 vx_v, vy_v, vz_v,
                 djx_v, djy_v, djz_v, dix_v, diy_v, diz_v,
                 cix_v, ciy_v, ciz_v, cjx_v, cjy_v, cjz_v,
                 gsem, ssem):
    cid = lax.axis_index("c")
    sid = lax.axis_index("s")
    s0 = pl.multiple_of(sid * SEG, 128)
    for c, dst in ((0, dsx), (1, dsy), (2, dsz)):
        pltpu.sync_copy(dip_h.at[pl.ds(pl.multiple_of(c * NPAD + s0, 128), SEG)],
                        dst.at[pl.ds(s0, SEG)])
    for dst in (asx, asy, asz):
        pltpu.sync_copy(zeros_h.at[pl.ds(s0, SEG)], dst.at[pl.ds(s0, SEG)])
    plsc.subcore_barrier()

    wid = sid * NC + cid

    def block(b, carry):
        blk = wid * NBLK + b
        e0 = blk * B
        r0 = blk * SUB
        pltpu.sync_copy(ii_h.at[pl.ds(r0, SUB)], ii_v)
        pltpu.sync_copy(jj_h.at[pl.ds(r0, SUB)], jj_v)
        pltpu.sync_copy(vx_h.at[pl.ds(e0, B)], vx_v)
        pltpu.sync_copy(vy_h.at[pl.ds(e0, B)], vy_v)
        pltpu.sync_copy(vz_h.at[pl.ds(e0, B)], vz_v)
        descs = []
        for k in range(SUB):
            w = pl.ds(k * 128, 128)
            for src, dst in ((dsx, djx_v), (dsy, djy_v), (dsz, djz_v)):
                descs.append(pltpu.async_copy(src.at[jj_v.at[k]], dst.at[w], gsem))
            for src, dst in ((dsx, dix_v), (dsy, diy_v), (dsz, diz_v)):
                descs.append(pltpu.async_copy(src.at[ii_v.at[k]], dst.at[w], gsem))
        for d in descs:
            d.wait()

        def vop(m, c2):
            o = m * L
            w = pl.ds(o, L)
            vx = vx_v[w]
            vy = vy_v[w]
            vz = vz_v[w]
            r2 = vx * vx + vy * vy + vz * vz
            bits = lax.bitcast_convert_type(r2, jnp.int32)
            y = lax.bitcast_convert_type(
                jnp.int32(0x5F3759DF) - (bits >> 1), jnp.float32)
            y = y * (1.5 - 0.5 * r2 * y * y)
            y = y * (1.5 - 0.5 * r2 * y * y)
            y = y * (1.5 - 0.5 * r2 * y * y)
            y2 = y * y
            sh = 0.5 * (y2 * y)
            th = 3.0 * (y2 * sh)
            djx = djx_v[w]
            djy = djy_v[w]
            djz = djz_v[w]
            aj = (djx * vx + djy * vy + djz * vz) * th
            cix_v[w] = djx * sh - vx * aj
            ciy_v[w] = djy * sh - vy * aj
            ciz_v[w] = djz * sh - vz * aj
            dix = dix_v[w]
            diy = diy_v[w]
            diz = diz_v[w]
            ai = (dix * vx + diy * vy + diz * vz) * th
            cjx_v[w] = dix * sh - vx * ai
            cjy_v[w] = diy * sh - vy * ai
            cjz_v[w] = diz * sh - vz * ai
            return c2

        lax.fori_loop(0, B // L, vop, 0)

        descs = []
        for k in range(SUB):
            w = pl.ds(k * 128, 128)
            for src, dst in ((cix_v, asx), (ciy_v, asy), (ciz_v, asz)):
                descs.append(
                    pltpu.async_copy(src.at[w], dst.at[ii_v.at[k]], ssem, add=True))
            for src, dst in ((cjx_v, asx), (cjy_v, asy), (cjz_v, asz)):
                descs.append(
                    pltpu.async_copy(src.at[w], dst.at[jj_v.at[k]], ssem, add=True))
        for d in descs:
            d.wait()
        return carry

    lax.fori_loop(0, NBLK, block, 0)
    plsc.subcore_barrier()
    for c, src in ((0, asx), (1, asy), (2, asz)):
        o = pl.multiple_of((cid * 3 + c) * NPAD + s0, 128)
        pltpu.sync_copy(src.at[pl.ds(s0, SEG)], out_h.at[pl.ds(o, SEG)])


@functools.partial(
    pl.kernel,
    out_type=jax.ShapeDtypeStruct((NPAD * 3,), jnp.float32),
    mesh=_mesh,
    compiler_params=_params,
    scratch_types=(
        [pltpu.VMEM((R2C,), jnp.float32) for _ in range(6)]
        + [pltpu.VMEM((W2C,), jnp.float32)]
    ),
)
def _combine_kernel(p_h, out_h, p0x, p0y, p0z, p1x, p1y, p1z, stage):
    cid = lax.axis_index("c")
    sid = lax.axis_index("s")
    wid = sid * NC + cid
    r0 = pl.multiple_of(wid * R2C, 128)
    bufs = (p0x, p0y, p0z, p1x, p1y, p1z)
    for g in range(NC):
        for c in range(3):
            o = pl.multiple_of((g * 3 + c) * NPAD + r0, 128)
            pltpu.sync_copy(p_h.at[pl.ds(o, R2C)], bufs[g * 3 + c])
    i3 = lax.broadcasted_iota(jnp.int32, (L,), 0) * 3

    def vop(m, carry):
        w = pl.ds(m * L, L)
        x = p0x[w] + p1x[w]
        y = p0y[w] + p1y[w]
        z = p0z[w] + p1z[w]
        base = m * (3 * L) + i3
        plsc.store_scatter(stage, [base], x)
        plsc.store_scatter(stage, [base + 1], y)
        plsc.store_scatter(stage, [base + 2], z)
        return carry

    lax.fori_loop(0, R2C // L, vop, 0)
    pltpu.sync_copy(stage, out_h.at[pl.ds(pl.multiple_of(wid * W2C, 128), W2C)])


def kernel(dipoles, cell, positions, neighbor_indices, neighbor_vectors):
    del cell, positions
    idx = neighbor_indices.astype(jnp.int32)
    nv = neighbor_vectors.astype(jnp.float32)
    pad = EPAD - E
    pad_idx = N + (jnp.arange(pad, dtype=jnp.int32) % (NPAD - N))
    ii = jnp.concatenate([idx[:, 0], pad_idx]).reshape(NROWS, 128)
    jj = jnp.concatenate([idx[:, 1], pad_idx]).reshape(NROWS, 128)
    vt = nv.T
    pz = jnp.zeros((pad,), jnp.float32)
    vx = jnp.concatenate([vt[0], pz + 1.0])
    vy = jnp.concatenate([vt[1], pz])
    vz = jnp.concatenate([vt[2], pz])
    dip = jnp.zeros((3, NPAD), jnp.float32).at[:, :N].set(
        dipoles.astype(jnp.float32).T).reshape(-1)
    zeros = jnp.zeros((NPAD,), jnp.float32)
    part = _edge_kernel(dip, zeros, ii, jj, vx, vy, vz)
    flat = _combine_kernel(part)
    return flat[: N * 3].reshape(N, 3)


# speedup vs baseline: 33.8984x; 1.0080x over previous
import functools

import jax
import jax.numpy as jnp
from jax import lax
from jax.experimental import pallas as pl
from jax.experimental.pallas import tpu as pltpu
from jax.experimental.pallas import tpu_sc as plsc

N = 50000
E = 1_600_000
NC, NS, L = 2, 16, 16
NW = NC * NS
NPAD = 53248
SEG = NPAD // NS
B = 1024
SUB = B // 128
NBLK = 50
PER_W = NBLK * B
EPAD = NW * PER_W
NROWS = EPAD // 128
R2C = NPAD // NW
W2C = R2C * 3

_mesh = plsc.VectorSubcoreMesh(core_axis_name="c", subcore_axis_name="s")
_params = pltpu.CompilerParams(needs_layout_passes=False)


@functools.partial(
    pl.kernel,
    out_type=jax.ShapeDtypeStruct((NC * 3 * NPAD,), jnp.float32),
    mesh=_mesh,
    compiler_params=_params,
    scratch_types=(
        [pltpu.VMEM_SHARED((NPAD,), jnp.float32) for _ in range(3)]
        + [pltpu.VMEM_SHARED((NPAD,), jnp.float32) for _ in range(3)]
        + [pltpu.VMEM((B,), jnp.int32) for _ in range(2)]
        + [pltpu.VMEM((B,), jnp.float32) for _ in range(3)]
        + [pltpu.VMEM((B,), jnp.float32) for _ in range(6)]
        + [pltpu.VMEM((B,), jnp.float32) for _ in range(6)]
        + [pltpu.SemaphoreType.DMA, pltpu.SemaphoreType.DMA]
    ),
)
def _edge_kernel(dip_h, zeros_h, ii_h, jj_h, vx_h, vy_h, vz_h, out_h,
                 dsx, dsy, dsz, asx, asy, asz, ii_v, jj_v,
                 vx_v, vy_v, vz_v,
                 djx_v, djy_v, djz_v, dix_v, diy_v, diz_v,
                 cix_v, ciy_v, ciz_v, cjx_v, cjy_v, cjz_v,
                 gsem, ssem):
    cid = lax.axis_index("c")
    sid = lax.axis_index("s")
    s0 = pl.multiple_of(sid * SEG, 128)
    for c, dst in ((0, dsx), (1, dsy), (2, dsz)):
        pltpu.sync_copy(dip_h.at[pl.ds(pl.multiple_of(c * NPAD + s0, 128), SEG)],
                        dst.at[pl.ds(s0, SEG)])
    for dst in (asx, asy, asz):
        pltpu.sync_copy(zeros_h.at[pl.ds(s0, SEG)], dst.at[pl.ds(s0, SEG)])
    plsc.subcore_barrier()

    wid = sid * NC + cid

    def block(b, carry):
        blk = wid * NBLK + b
        e0 = blk * B
        r0 = blk * SUB
        pltpu.sync_copy(ii_h.at[pl.ds(e0, B)], ii_v)
        pltpu.sync_copy(jj_h.at[pl.ds(e0, B)], jj_v)
        pltpu.sync_copy(vx_h.at[pl.ds(e0, B)], vx_v)
        pltpu.sync_copy(vy_h.at[pl.ds(e0, B)], vy_v)
        pltpu.sync_copy(vz_h.at[pl.ds(e0, B)], vz_v)
        descs = []
        for src, dst in ((dsx, djx_v), (dsy, djy_v), (dsz, djz_v)):
            descs.append(pltpu.async_copy(src.at[jj_v], dst, gsem))
        for src, dst in ((dsx, dix_v), (dsy, diy_v), (dsz, diz_v)):
            descs.append(pltpu.async_copy(src.at[ii_v], dst, gsem))
        for d in descs:
            d.wait()

        def vop(m, c2):
            o = m * L
            w = pl.ds(o, L)
            vx = vx_v[w]
            vy = vy_v[w]
            vz = vz_v[w]
            r2 = vx * vx + vy * vy + vz * vz
            bits = lax.bitcast_convert_type(r2, jnp.int32)
            y = lax.bitcast_convert_type(
                jnp.int32(0x5F3759DF) - (bits >> 1), jnp.float32)
            y = y * (1.5 - 0.5 * r2 * y * y)
            y = y * (1.5 - 0.5 * r2 * y * y)
            y = y * (1.5 - 0.5 * r2 * y * y)
            y2 = y * y
            sh = 0.5 * (y2 * y)
            th = 3.0 * (y2 * sh)
            djx = djx_v[w]
            djy = djy_v[w]
            djz = djz_v[w]
            aj = (djx * vx + djy * vy + djz * vz) * th
            cix_v[w] = djx * sh - vx * aj
            ciy_v[w] = djy * sh - vy * aj
            ciz_v[w] = djz * sh - vz * aj
            dix = dix_v[w]
            diy = diy_v[w]
            diz = diz_v[w]
            ai = (dix * vx + diy * vy + diz * vz) * th
            cjx_v[w] = dix * sh - vx * ai
            cjy_v[w] = diy * sh - vy * ai
            cjz_v[w] = diz * sh - vz * ai
            return c2

        lax.fori_loop(0, B // L, vop, 0)

        descs = []
        for src, dst in ((cix_v, asx), (ciy_v, asy), (ciz_v, asz)):
            descs.append(pltpu.async_copy(src, dst.at[ii_v], ssem, add=True))
        for src, dst in ((cjx_v, asx), (cjy_v, asy), (cjz_v, asz)):
            descs.append(pltpu.async_copy(src, dst.at[jj_v], ssem, add=True))
        for d in descs:
            d.wait()
        return carry

    lax.fori_loop(0, NBLK, block, 0)
    plsc.subcore_barrier()
    for c, src in ((0, asx), (1, asy), (2, asz)):
        o = pl.multiple_of((cid * 3 + c) * NPAD + s0, 128)
        pltpu.sync_copy(src.at[pl.ds(s0, SEG)], out_h.at[pl.ds(o, SEG)])


@functools.partial(
    pl.kernel,
    out_type=jax.ShapeDtypeStruct((NPAD * 3,), jnp.float32),
    mesh=_mesh,
    compiler_params=_params,
    scratch_types=(
        [pltpu.VMEM((R2C,), jnp.float32) for _ in range(6)]
        + [pltpu.VMEM((W2C,), jnp.float32)]
    ),
)
def _combine_kernel(p_h, out_h, p0x, p0y, p0z, p1x, p1y, p1z, stage):
    cid = lax.axis_index("c")
    sid = lax.axis_index("s")
    wid = sid * NC + cid
    r0 = pl.multiple_of(wid * R2C, 128)
    bufs = (p0x, p0y, p0z, p1x, p1y, p1z)
    for g in range(NC):
        for c in range(3):
            o = pl.multiple_of((g * 3 + c) * NPAD + r0, 128)
            pltpu.sync_copy(p_h.at[pl.ds(o, R2C)], bufs[g * 3 + c])
    i3 = lax.broadcasted_iota(jnp.int32, (L,), 0) * 3

    def vop(m, carry):
        w = pl.ds(m * L, L)
        x = p0x[w] + p1x[w]
        y = p0y[w] + p1y[w]
        z = p0z[w] + p1z[w]
        base = m * (3 * L) + i3
        plsc.store_scatter(stage, [base], x)
        plsc.store_scatter(stage, [base + 1], y)
        plsc.store_scatter(stage, [base + 2], z)
        return carry

    lax.fori_loop(0, R2C // L, vop, 0)
    pltpu.sync_copy(stage, out_h.at[pl.ds(pl.multiple_of(wid * W2C, 128), W2C)])


def kernel(dipoles, cell, positions, neighbor_indices, neighbor_vectors):
    del cell, positions
    idx = neighbor_indices.astype(jnp.int32)
    nv = neighbor_vectors.astype(jnp.float32)
    pad = EPAD - E
    pad_idx = N + (jnp.arange(pad, dtype=jnp.int32) % (NPAD - N))
    ii = jnp.concatenate([idx[:, 0], pad_idx])
    jj = jnp.concatenate([idx[:, 1], pad_idx])
    vt = nv.T
    pz = jnp.zeros((pad,), jnp.float32)
    vx = jnp.concatenate([vt[0], pz + 1.0])
    vy = jnp.concatenate([vt[1], pz])
    vz = jnp.concatenate([vt[2], pz])
    dip = jnp.zeros((3, NPAD), jnp.float32).at[:, :N].set(
        dipoles.astype(jnp.float32).T).reshape(-1)
    zeros = jnp.zeros((NPAD,), jnp.float32)
    part = _edge_kernel(dip, zeros, ii, jj, vx, vy, vz)
    flat = _combine_kernel(part)
    return flat[: N * 3].reshape(N, 3)
